# baseline (speedup 1.0000x reference)
import functools

import jax
import jax.numpy as jnp
from jax import lax
from jax.experimental import pallas as pl
from jax.experimental.pallas import tpu as pltpu
from jax.experimental.pallas import tpu_sc as plsc

VOCAB = 1000
EMBED = 64
BATCH = 16384

NC = 2
NS = 16
NW = NC * NS
B_PER_W = BATCH // NW
CHUNK = 128
NCHUNK = B_PER_W // CHUNK
LANES = 16
EB = EMBED // 8
BB = BATCH // CHUNK

_MESH = plsc.VectorSubcoreMesh(core_axis_name="c", subcore_axis_name="s")


@functools.partial(
    pl.kernel,
    mesh=_MESH,
    out_type=jax.ShapeDtypeStruct((EB, BB, 8, CHUNK), jnp.float32),
    scratch_types=[
        pltpu.VMEM((B_PER_W,), jnp.int32),
        pltpu.VMEM((B_PER_W,), jnp.float32),
        pltpu.VMEM((B_PER_W, EMBED), jnp.float32),
        pltpu.VMEM((NCHUNK, EB, 8, CHUNK), jnp.float32),
        pltpu.SemaphoreType.DMA,
        pltpu.SemaphoreType.DMA,
    ],
    compiler_params=pltpu.CompilerParams(
        use_tc_tiling_on_sc=False, needs_layout_passes=False
    ),
)
def _embed_lookup(idx_hbm, table_hbm, out_hbm, idx_v, mask_v, rows_v, blk_v, gsem, osem):
    wid = lax.axis_index("s") * NC + lax.axis_index("c")
    base = wid * B_PER_W
    pltpu.sync_copy(idx_hbm.at[pl.ds(base, B_PER_W)], idx_v)
    gathers = []
    for j in range(NCHUNK):
        for i in range(CHUNK // LANES):
            v = idx_v[pl.ds(j * CHUNK + i * LANES, LANES)]
            mask_v[pl.ds(j * CHUNK + i * LANES, LANES)] = jnp.where(
                v == 0, 0.0, 1.0
            )
        gathers.append(
            pltpu.async_copy(
                table_hbm.at[idx_v.at[pl.ds(j * CHUNK, CHUNK)]],
                rows_v.at[pl.ds(j * CHUNK, CHUNK)],
                gsem,
            )
        )
    lane = lax.broadcasted_iota(jnp.int32, (LANES,), 0)
    writes = []
    for j in range(NCHUNK):
        gathers[j].wait()
        mvecs = [mask_v[pl.ds(j * CHUNK + g * LANES, LANES)] for g in range(8)]
        rowidx = [lane + (j * CHUNK + g * LANES) for g in range(8)]
        def body(es, carry, j=j):
            for eb in range(EB):
                colidx = jnp.full((LANES,), eb * 8, jnp.int32) + es
                for g in range(8):
                    vec = plsc.load_gather(rows_v, [rowidx[g], colidx])
                    blk_v[j, eb, es, pl.ds(g * LANES, LANES)] = vec * mvecs[g]
            return carry

        lax.fori_loop(0, 8, body, jnp.int32(0))
        writes.append(
            pltpu.async_copy(
                blk_v.at[j],
                out_hbm.at[:, wid * NCHUNK + j],
                osem,
            )
        )
    for c in writes:
        c.wait()


def kernel(age, table):
    arr4 = _embed_lookup(age.astype(jnp.int32), table)
    return arr4.transpose(1, 3, 0, 2).reshape(BATCH, EMBED)


# speedup vs baseline: 1.2394x; 1.2394x over previous
import functools

import jax
import jax.numpy as jnp
from jax import lax
from jax.experimental import pallas as pl
from jax.experimental.pallas import tpu as pltpu
from jax.experimental.pallas import tpu_sc as plsc

VOCAB = 1000
EMBED = 64
BATCH = 16384
HALF = BATCH // 2

NC = 2
NS = 16
NW = NC * NS
B_PER_W = HALF // NW
CHUNK = 128
NCHUNK = B_PER_W // CHUNK
LANES = 16

_MESH = plsc.VectorSubcoreMesh(core_axis_name="c", subcore_axis_name="s")


@functools.partial(
    pl.kernel,
    mesh=_MESH,
    out_type=jax.ShapeDtypeStruct((HALF, 2 * EMBED), jnp.float32),
    scratch_types=[
        pltpu.VMEM((B_PER_W,), jnp.int32),
        pltpu.VMEM((B_PER_W,), jnp.float32),
        pltpu.VMEM((B_PER_W, EMBED), jnp.float32),
        pltpu.SemaphoreType.DMA,
        pltpu.SemaphoreType.DMA,
    ],
    compiler_params=pltpu.CompilerParams(
        use_tc_tiling_on_sc=False, needs_layout_passes=False
    ),
)
def _embed_lookup(idx_hbm, table_hbm, out_hbm, idx_v, mask_v, rows_v, gsem, osem):
    wid = lax.axis_index("s") * NC + lax.axis_index("c")
    base = wid * B_PER_W
    pltpu.sync_copy(idx_hbm.at[pl.ds(base, B_PER_W)], idx_v)
    gathers = []
    counts = []
    for j in range(NCHUNK):
        cnt = jnp.int32(0)
        for i in range(CHUNK // LANES):
            v = idx_v[pl.ds(j * CHUNK + i * LANES, LANES)]
            z = v == 0
            mask_v[pl.ds(j * CHUNK + i * LANES, LANES)] = jnp.where(z, 0.0, 1.0)
            cnt = cnt + jnp.sum(jnp.where(z, 1, 0))
        counts.append(cnt)
        gathers.append(
            pltpu.async_copy(
                table_hbm.at[idx_v.at[pl.ds(j * CHUNK, CHUNK)]],
                rows_v.at[pl.ds(j * CHUNK, CHUNK)],
                gsem,
            )
        )
    writes = []
    for j in range(NCHUNK):
        gathers[j].wait()

        @pl.when(counts[j] > 0)
        def _fixup(j=j):
            def body(r, carry):
                m = plsc.load_gather(mask_v, [jnp.full((LANES,), r, jnp.int32)])
                for d in range(EMBED // LANES):
                    rows_v[r, pl.ds(d * LANES, LANES)] = (
                        rows_v[r, pl.ds(d * LANES, LANES)] * m
                    )
                return carry

            lax.fori_loop(j * CHUNK, (j + 1) * CHUNK, body, jnp.int32(0))

        writes.append(
            pltpu.async_copy(
                rows_v.at[pl.ds(j * CHUNK, CHUNK)],
                out_hbm.at[pl.ds(base + j * CHUNK, CHUNK), pl.ds(0, EMBED)],
                osem,
            )
        )
    for c in writes:
        c.wait()


def kernel(age, table):
    idx = age.astype(jnp.int32)
    oa = _embed_lookup(idx[:HALF], table)
    ob = _embed_lookup(idx[HALF:], table)
    return jnp.concatenate([oa[:, :EMBED], ob[:, :EMBED]], axis=0)


# speedup vs baseline: 1.7124x; 1.3816x over previous
import functools

import jax
import jax.numpy as jnp
from jax import lax
from jax.experimental import pallas as pl
from jax.experimental.pallas import tpu as pltpu
from jax.experimental.pallas import tpu_sc as plsc

VOCAB = 1000
EMBED = 64
BATCH = 16384

NC = 2
NS = 16
NW = NC * NS
B_PER_W = BATCH // NW
CHUNK = 128
NCHUNK = B_PER_W // CHUNK

_MESH = plsc.VectorSubcoreMesh(core_axis_name="c", subcore_axis_name="s")


@functools.partial(
    pl.kernel,
    mesh=_MESH,
    out_type=jax.ShapeDtypeStruct((BATCH, 2 * EMBED), jnp.float32),
    scratch_types=[
        pltpu.VMEM((B_PER_W,), jnp.int32),
        pltpu.VMEM((B_PER_W, EMBED), jnp.float32),
        pltpu.SemaphoreType.DMA,
        pltpu.SemaphoreType.DMA,
    ],
    compiler_params=pltpu.CompilerParams(
        use_tc_tiling_on_sc=False, needs_layout_passes=False
    ),
)
def _embed_lookup(idx_hbm, table_hbm, out_hbm, idx_v, rows_v, gsem, osem):
    wid = lax.axis_index("s") * NC + lax.axis_index("c")
    base = wid * B_PER_W
    pltpu.sync_copy(idx_hbm.at[pl.ds(base, B_PER_W)], idx_v)
    gathers = [
        pltpu.async_copy(
            table_hbm.at[idx_v.at[pl.ds(j * CHUNK, CHUNK)]],
            rows_v.at[pl.ds(j * CHUNK, CHUNK)],
            gsem,
        )
        for j in range(NCHUNK)
    ]
    writes = []
    for j in range(NCHUNK):
        gathers[j].wait()
        writes.append(
            pltpu.async_copy(
                rows_v.at[pl.ds(j * CHUNK, CHUNK)],
                out_hbm.at[pl.ds(base + j * CHUNK, CHUNK), pl.ds(0, EMBED)],
                osem,
            )
        )
    for c in writes:
        c.wait()


def kernel(age, table):
    t = table.at[0].set(0.0)
    out = _embed_lookup(age.astype(jnp.int32), t)
    return out[:, :EMBED]


# speedup vs baseline: 1.7232x; 1.0063x over previous
import functools

import jax
import jax.numpy as jnp
from jax import lax
from jax.experimental import pallas as pl
from jax.experimental.pallas import tpu as pltpu
from jax.experimental.pallas import tpu_sc as plsc

VOCAB = 1000
EMBED = 64
BATCH = 16384

NC = 2
NS = 16
NW = NC * NS
B_PER_W = BATCH // NW
CHUNK = 64
NCHUNK = B_PER_W // CHUNK

_MESH = plsc.VectorSubcoreMesh(core_axis_name="c", subcore_axis_name="s")


@functools.partial(
    pl.kernel,
    mesh=_MESH,
    out_type=jax.ShapeDtypeStruct((BATCH, 2 * EMBED), jnp.float32),
    scratch_types=[
        pltpu.VMEM((B_PER_W,), jnp.int32),
        pltpu.VMEM((B_PER_W, EMBED), jnp.float32),
        pltpu.SemaphoreType.DMA,
        pltpu.SemaphoreType.DMA,
    ],
    compiler_params=pltpu.CompilerParams(
        use_tc_tiling_on_sc=False, needs_layout_passes=False
    ),
)
def _embed_lookup(idx_hbm, table_hbm, out_hbm, idx_v, rows_v, gsem, osem):
    wid = lax.axis_index("s") * NC + lax.axis_index("c")
    base = wid * B_PER_W
    pltpu.sync_copy(idx_hbm.at[pl.ds(base, B_PER_W)], idx_v)
    gathers = [
        pltpu.async_copy(
            table_hbm.at[idx_v.at[pl.ds(j * CHUNK, CHUNK)]],
            rows_v.at[pl.ds(j * CHUNK, CHUNK)],
            gsem,
        )
        for j in range(NCHUNK)
    ]
    writes = []
    for j in range(NCHUNK):
        gathers[j].wait()
        writes.append(
            pltpu.async_copy(
                rows_v.at[pl.ds(j * CHUNK, CHUNK)],
                out_hbm.at[pl.ds(base + j * CHUNK, CHUNK), pl.ds(0, EMBED)],
                osem,
            )
        )
    for c in writes:
        c.wait()


def kernel(age, table):
    t = table.at[0].set(0.0)
    out = _embed_lookup(age.astype(jnp.int32), t)
    return out[:, :EMBED]


# speedup vs baseline: 1.7326x; 1.0054x over previous
import functools

import jax
import jax.numpy as jnp
from jax import lax
from jax.experimental import pallas as pl
from jax.experimental.pallas import tpu as pltpu
from jax.experimental.pallas import tpu_sc as plsc

VOCAB = 1000
EMBED = 64
BATCH = 16384

NC = 2
NS = 16
NW = NC * NS
B_PER_W = BATCH // NW
CHUNK = 128
NCHUNK = B_PER_W // CHUNK

_MESH = plsc.VectorSubcoreMesh(core_axis_name="c", subcore_axis_name="s")


@functools.partial(
    pl.kernel,
    mesh=_MESH,
    out_type=jax.ShapeDtypeStruct((BATCH, 2 * EMBED), jnp.float32),
    scratch_types=[
        pltpu.VMEM((B_PER_W,), jnp.int32),
        pltpu.VMEM((B_PER_W, EMBED), jnp.float32),
        pltpu.SemaphoreType.DMA,
        pltpu.SemaphoreType.DMA,
    ],
    compiler_params=pltpu.CompilerParams(
        use_tc_tiling_on_sc=False, needs_layout_passes=False
    ),
)
def _embed_lookup(idx_hbm, table_hbm, out_hbm, idx_v, rows_v, gsem, osem):
    wid = lax.axis_index("s") * NC + lax.axis_index("c")
    base = wid * B_PER_W
    pltpu.sync_copy(idx_hbm.at[pl.ds(base, B_PER_W)], idx_v)
    gathers = [
        pltpu.async_copy(
            table_hbm.at[idx_v.at[pl.ds(j * CHUNK, CHUNK)]],
            rows_v.at[pl.ds(j * CHUNK, CHUNK)],
            gsem,
        )
        for j in range(NCHUNK)
    ]
    writes = []
    for j in range(NCHUNK):
        gathers[j].wait()
        writes.append(
            pltpu.async_copy(
                rows_v.at[pl.ds(j * CHUNK, CHUNK)],
                out_hbm.at[pl.ds(base + j * CHUNK, CHUNK), pl.ds(0, EMBED)],
                osem,
            )
        )
    for c in writes:
        c.wait()


def kernel(age, table):
    t = table.at[0].set(0.0)
    out = _embed_lookup(age.astype(jnp.int32), t)
    return out[:, :EMBED]
